# baseline (device time: 93093 ns/iter reference)
import jax
import jax.numpy as jnp
from jax import lax
from jax.experimental import pallas as pl
from jax.experimental.pallas import tpu as pltpu

P = 8


def kernel(x, w_mat):
    m_total, k_loc = x.shape
    k_total, n = w_mat.shape
    m_loc = m_total // P

    def body(x_ref, w_ref, o_ref, comm_ref, send_sems, recv_sems):
        me = lax.axis_index("i")

        barrier = pltpu.get_barrier_semaphore()
        for j in range(P):
            @pl.when(j != me)
            def _():
                pl.semaphore_signal(
                    barrier, inc=1,
                    device_id=(j,), device_id_type=pl.DeviceIdType.MESH,
                )
        pl.semaphore_wait(barrier, P - 1)

        for t in range(1, P):
            j = (me + t) % P
            pltpu.make_async_remote_copy(
                src_ref=x_ref.at[pl.ds(j * m_loc, m_loc), :],
                dst_ref=comm_ref.at[me],
                send_sem=send_sems.at[j],
                recv_sem=recv_sems.at[me],
                device_id=(j,),
                device_id_type=pl.DeviceIdType.MESH,
            ).start()

        o_ref[:, :] = jnp.dot(
            x_ref[pl.ds(me * m_loc, m_loc), :],
            w_ref[pl.ds(me * k_loc, k_loc), :],
            preferred_element_type=jnp.float32,
        )

        for t in range(1, P):
            j = (me - t) % P
            pltpu.make_async_remote_copy(
                src_ref=x_ref.at[pl.ds(0, m_loc), :],
                dst_ref=comm_ref.at[j],
                send_sem=send_sems.at[j],
                recv_sem=recv_sems.at[j],
                device_id=(j,),
                device_id_type=pl.DeviceIdType.MESH,
            ).wait_recv()
            o_ref[:, :] += jnp.dot(
                comm_ref[j],
                w_ref[pl.ds(j * k_loc, k_loc), :],
                preferred_element_type=jnp.float32,
            )

        o_ref[:, :] = jnp.maximum(o_ref[:, :], 0.0)

        for t in range(1, P):
            j = (me + t) % P
            pltpu.make_async_remote_copy(
                src_ref=x_ref.at[pl.ds(j * m_loc, m_loc), :],
                dst_ref=comm_ref.at[j],
                send_sem=send_sems.at[j],
                recv_sem=recv_sems.at[j],
                device_id=(j,),
                device_id_type=pl.DeviceIdType.MESH,
            ).wait_send()

    return pl.pallas_call(
        body,
        out_shape=jax.ShapeDtypeStruct((m_loc, n), jnp.float32),
        in_specs=[
            pl.BlockSpec(memory_space=pltpu.VMEM),
            pl.BlockSpec(memory_space=pltpu.VMEM),
        ],
        out_specs=pl.BlockSpec(memory_space=pltpu.VMEM),
        scratch_shapes=[
            pltpu.VMEM((P, m_loc, k_loc), jnp.float32),
            pltpu.SemaphoreType.DMA((P,)),
            pltpu.SemaphoreType.DMA((P,)),
        ],
        compiler_params=pltpu.CompilerParams(
            collective_id=0,
            vmem_limit_bytes=100 * 1024 * 1024,
        ),
    )(x, w_mat)


# device time: 59476 ns/iter; 1.5652x vs baseline; 1.5652x over previous
import jax
import jax.numpy as jnp
from jax import lax
from jax.experimental import pallas as pl
from jax.experimental.pallas import tpu as pltpu

P = 8


def kernel(x, w_mat):
    m_total, k_loc = x.shape
    k_total, n = w_mat.shape
    m_loc = m_total // P

    def body(x_ref, w_ref, o_ref, xb_ref, comm_ref,
             send_sems, recv_sems):
        me = lax.axis_index("i")

        xb_ref[:, :] = x_ref[:, :].astype(jnp.bfloat16)

        barrier = pltpu.get_barrier_semaphore()
        for j in range(P):
            @pl.when(j != me)
            def _():
                pl.semaphore_signal(
                    barrier, inc=1,
                    device_id=(j,), device_id_type=pl.DeviceIdType.MESH,
                )
        pl.semaphore_wait(barrier, P - 1)

        for t in range(1, P):
            j = (me + t) % P
            pltpu.make_async_remote_copy(
                src_ref=xb_ref.at[pl.ds(j * m_loc, m_loc), :],
                dst_ref=comm_ref.at[me],
                send_sem=send_sems.at[j],
                recv_sem=recv_sems.at[me],
                device_id=(j,),
                device_id_type=pl.DeviceIdType.MESH,
            ).start()

        o_ref[:, :] = jnp.dot(
            xb_ref[pl.ds(me * m_loc, m_loc), :],
            w_ref[pl.ds(me * k_loc, k_loc), :],
            preferred_element_type=jnp.float32,
        )

        for t in range(1, P):
            j = (me - t) % P
            pltpu.make_async_remote_copy(
                src_ref=xb_ref.at[pl.ds(0, m_loc), :],
                dst_ref=comm_ref.at[j],
                send_sem=send_sems.at[j],
                recv_sem=recv_sems.at[j],
                device_id=(j,),
                device_id_type=pl.DeviceIdType.MESH,
            ).wait_recv()
            o_ref[:, :] += jnp.dot(
                comm_ref[j],
                w_ref[pl.ds(j * k_loc, k_loc), :],
                preferred_element_type=jnp.float32,
            )

        o_ref[:, :] = jnp.maximum(o_ref[:, :], 0.0)

        for t in range(1, P):
            j = (me + t) % P
            pltpu.make_async_remote_copy(
                src_ref=xb_ref.at[pl.ds(j * m_loc, m_loc), :],
                dst_ref=comm_ref.at[j],
                send_sem=send_sems.at[j],
                recv_sem=recv_sems.at[j],
                device_id=(j,),
                device_id_type=pl.DeviceIdType.MESH,
            ).wait_send()

    return pl.pallas_call(
        body,
        out_shape=jax.ShapeDtypeStruct((m_loc, n), jnp.float32),
        in_specs=[
            pl.BlockSpec(memory_space=pltpu.VMEM),
            pl.BlockSpec(memory_space=pltpu.VMEM),
        ],
        out_specs=pl.BlockSpec(memory_space=pltpu.VMEM),
        scratch_shapes=[
            pltpu.VMEM((m_total, k_loc), jnp.bfloat16),
            pltpu.VMEM((P, m_loc, k_loc), jnp.bfloat16),
            pltpu.SemaphoreType.DMA((P,)),
            pltpu.SemaphoreType.DMA((P,)),
        ],
        compiler_params=pltpu.CompilerParams(
            collective_id=0,
            vmem_limit_bytes=110 * 1024 * 1024,
        ),
    )(x, w_mat)
